# baseline (speedup 1.0000x reference)
import jax
import jax.numpy as jnp
from jax.experimental import pallas as pl
from jax.experimental.pallas import tpu as pltpu

N = 50000
F_IN = 128
D = 64
T = 1000
H = 2

BN = 1000
TP = 1024


def _leaky_relu(x, slope=0.2):
    return jnp.where(x > 0, x, slope * x)


def _gat_kernel(ids_ref, adj_ref, emb_ref, w0_ref, as0_ref, ad0_ref,
                w1_ref, as1_ref, ad1_ref, h_out_ref):
    adj = adj_ref[...]
    ids = ids_ref[...]
    iota = jax.lax.broadcasted_iota(jnp.int32, (T, T), 1)
    one_hot = (ids == iota).astype(jnp.float32)
    te = jax.lax.dot_general(one_hot, emb_ref[...],
                             (((1,), (0,)), ((), ())),
                             preferred_element_type=jnp.float32)

    def layer(h_in, w_ref, asrc_ref, adst_ref):
        acc = jnp.zeros((T, D), dtype=jnp.float32)
        for head in range(H):
            hh = h_in * w_ref[head, :][None, :]
            f_src = jnp.sum(hh * asrc_ref[head, :][None, :], axis=1,
                            keepdims=True)
            f_dst = jnp.sum(hh * adst_ref[head, :][None, :], axis=1,
                            keepdims=True)
            e = f_src + f_dst.T
            e = _leaky_relu(e)
            e = jnp.where(adj > 0, e, jnp.float32(-1e9))
            m = jnp.max(e, axis=1, keepdims=True)
            p = jnp.exp(e - m)
            s = jnp.sum(p, axis=1, keepdims=True)
            a = p / s
            acc = acc + jax.lax.dot_general(
                a, hh, (((1,), (0,)), ((), ())),
                preferred_element_type=jnp.float32)
        return acc * jnp.float32(1.0 / H)

    h = layer(te, w0_ref, as0_ref, ad0_ref)
    h = jnp.where(h > 0, h, jnp.exp(h) - 1.0)
    h = layer(h, w1_ref, as1_ref, ad1_ref)
    h_out_ref[...] = h


def _fused_kernel(x_ref, w1_ref, b1_ref, hp_ref, out_ref):
    ent = jax.lax.dot_general(x_ref[...], w1_ref[...],
                              (((1,), (1,)), ((), ())),
                              preferred_element_type=jnp.float32)
    ent = jnp.maximum(ent + b1_ref[...], 0.0)
    out_ref[...] = jax.lax.dot_general(ent, hp_ref[...],
                                       (((1,), (1,)), ((), ())),
                                       preferred_element_type=jnp.float32)


def _gat(type_ids, type_adj, emb_table, gw0, ga_src0, ga_dst0, gw1, ga_src1, ga_dst1):
    ids2d = type_ids.reshape(T, 1)
    args = (ids2d, type_adj, emb_table,
            gw0.reshape(H, D), ga_src0.reshape(H, D), ga_dst0.reshape(H, D),
            gw1.reshape(H, D), ga_src1.reshape(H, D), ga_dst1.reshape(H, D))
    return pl.pallas_call(
        _gat_kernel,
        out_shape=jax.ShapeDtypeStruct((T, D), jnp.float32),
    )(*args)


def _fused(x, W1, b1, h):
    hp = jnp.pad(h, ((0, TP - T), (0, 0)))
    return pl.pallas_call(
        _fused_kernel,
        grid=(N // BN,),
        in_specs=[
            pl.BlockSpec((BN, F_IN), lambda i: (i, 0)),
            pl.BlockSpec((D, F_IN), lambda i: (0, 0)),
            pl.BlockSpec((1, D), lambda i: (0, 0)),
            pl.BlockSpec((TP, D), lambda i: (0, 0)),
        ],
        out_specs=pl.BlockSpec((BN, TP), lambda i: (i, 0)),
        out_shape=jax.ShapeDtypeStruct((N, T), jnp.float32),
    )(x, W1, b1.reshape(1, D), hp)


@jax.jit
def kernel(x, type_ids, type_adj, W1, b1, emb_table, gw0, ga_src0, ga_dst0,
           gw1, ga_src1, ga_dst1):
    h = _gat(type_ids, type_adj, emb_table, gw0, ga_src0, ga_dst0,
             gw1, ga_src1, ga_dst1)
    return _fused(x, W1, b1, h)


# speedup vs baseline: 1.0218x; 1.0218x over previous
import jax
import jax.numpy as jnp
from jax.experimental import pallas as pl
from jax.experimental.pallas import tpu as pltpu

N = 50000
F_IN = 128
D = 64
T = 1000
H = 2

BN = 1000
TP = 1024


def _leaky_relu(x, slope=0.2):
    return jnp.where(x > 0, x, slope * x)


def _gat_kernel(ids_ref, adj_ref, emb_ref, w0_ref, as0_ref, ad0_ref,
                w1_ref, as1_ref, ad1_ref, h_out_ref):
    adj = adj_ref[...]
    ids = ids_ref[...]
    iota = jax.lax.broadcasted_iota(jnp.int32, (T, T), 1)
    one_hot = (ids == iota).astype(jnp.float32)
    te = jax.lax.dot_general(one_hot, emb_ref[...],
                             (((1,), (0,)), ((), ())),
                             preferred_element_type=jnp.float32)

    def layer(h_in, w_ref, asrc_ref, adst_ref):
        acc = jnp.zeros((T, D), dtype=jnp.float32)
        for head in range(H):
            hh = h_in * w_ref[head, :][None, :]
            f_src = jnp.sum(hh * asrc_ref[head, :][None, :], axis=1,
                            keepdims=True)
            f_dst = jnp.sum(hh * adst_ref[head, :][None, :], axis=1,
                            keepdims=True)
            e = f_src + f_dst.T
            e = _leaky_relu(e)
            e = jnp.where(adj > 0, e, jnp.float32(-1e9))
            m = jnp.max(e, axis=1, keepdims=True)
            p = jnp.exp(e - m)
            s = jnp.sum(p, axis=1, keepdims=True)
            a = p / s
            acc = acc + jax.lax.dot_general(
                a, hh, (((1,), (0,)), ((), ())),
                preferred_element_type=jnp.float32)
        return acc * jnp.float32(1.0 / H)

    h = layer(te, w0_ref, as0_ref, ad0_ref)
    h = jnp.where(h > 0, h, jnp.exp(h) - 1.0)
    h = layer(h, w1_ref, as1_ref, ad1_ref)
    h_out_ref[...] = h


NSTEP = N // BN
NBUF = 4
CHUNKS = ((0, 256), (256, 256), (512, 256), (768, 232))


def _fused_kernel(x_ref, w1_ref, b1_ref, hp_ref, out_ref, *rest):
    accs = rest[:len(CHUNKS)]
    sems = rest[len(CHUNKS)]
    i = pl.program_id(0)
    s = jax.lax.rem(i, NBUF)
    rows = pl.ds(i * BN, BN)

    @pl.when(i >= NBUF)
    def _():
        for k, (c0, w) in enumerate(CHUNKS):
            pltpu.make_async_copy(accs[k].at[s], out_ref.at[rows, pl.ds(c0, w)],
                                  sems.at[k, s]).wait()

    ent = jax.lax.dot_general(x_ref[...], w1_ref[...],
                              (((1,), (1,)), ((), ())),
                              preferred_element_type=jnp.float32)
    ent = jnp.maximum(ent + b1_ref[...], 0.0)
    logits = jax.lax.dot_general(ent, hp_ref[...],
                                 (((1,), (1,)), ((), ())),
                                 preferred_element_type=jnp.float32)
    for k, (c0, w) in enumerate(CHUNKS):
        accs[k][s] = logits[:, c0:c0 + w]

    for k, (c0, w) in enumerate(CHUNKS):
        pltpu.make_async_copy(accs[k].at[s], out_ref.at[rows, pl.ds(c0, w)],
                              sems.at[k, s]).start()

    @pl.when(i == NSTEP - 1)
    def _():
        for k, (c0, w) in enumerate(CHUNKS):
            for b in range(NBUF):
                pltpu.make_async_copy(accs[k].at[b],
                                      out_ref.at[rows, pl.ds(c0, w)],
                                      sems.at[k, b]).wait()


def _gat(type_ids, type_adj, emb_table, gw0, ga_src0, ga_dst0, gw1, ga_src1, ga_dst1):
    ids2d = type_ids.reshape(T, 1)
    args = (ids2d, type_adj, emb_table,
            gw0.reshape(H, D), ga_src0.reshape(H, D), ga_dst0.reshape(H, D),
            gw1.reshape(H, D), ga_src1.reshape(H, D), ga_dst1.reshape(H, D))
    return pl.pallas_call(
        _gat_kernel,
        out_shape=jax.ShapeDtypeStruct((T, D), jnp.float32),
    )(*args)


def _fused(x, W1, b1, h):
    hp = jnp.pad(h, ((0, TP - T), (0, 0)))
    return pl.pallas_call(
        _fused_kernel,
        grid=(N // BN,),
        in_specs=[
            pl.BlockSpec((BN, F_IN), lambda i: (i, 0)),
            pl.BlockSpec((D, F_IN), lambda i: (0, 0)),
            pl.BlockSpec((1, D), lambda i: (0, 0)),
            pl.BlockSpec((TP, D), lambda i: (0, 0)),
        ],
        out_specs=pl.BlockSpec(memory_space=pltpu.HBM),
        out_shape=jax.ShapeDtypeStruct((N, T), jnp.float32),
        scratch_shapes=(
            [pltpu.VMEM((NBUF, BN, w), jnp.float32) for _, w in CHUNKS]
            + [pltpu.SemaphoreType.DMA((len(CHUNKS), NBUF))]
        ),
        compiler_params=pltpu.CompilerParams(
            dimension_semantics=("arbitrary",),
        ),
    )(x, W1, b1.reshape(1, D), hp)


@jax.jit
def kernel(x, type_ids, type_adj, W1, b1, emb_table, gw0, ga_src0, ga_dst0,
           gw1, ga_src1, ga_dst1):
    h = _gat(type_ids, type_adj, emb_table, gw0, ga_src0, ga_dst0,
             gw1, ga_src1, ga_dst1)
    return _fused(x, W1, b1, h)
